# baseline (speedup 1.0000x reference)
import functools

import jax
import jax.numpy as jnp
from jax import lax
from jax.experimental import pallas as pl
from jax.experimental.pallas import tpu as pltpu
from jax.experimental.pallas import tpu_sc as plsc

DIM = 128
CHUNK = 80
LANES = 16
QBLK = 32
RING = 5
DEPTH = 3


@functools.lru_cache(maxsize=None)
def _build(total):
    info = plsc.get_sparse_core_info()
    nc, ns = info.num_cores, info.num_subcores
    nw = nc * ns
    b_per_w = total // nw
    n_chunks = b_per_w // CHUNK
    assert total % (nw * CHUNK) == 0 and n_chunks % QBLK == 0
    assert n_chunks % RING == 0 and RING >= DEPTH + 2

    mesh = plsc.VectorSubcoreMesh(core_axis_name="c", subcore_axis_name="s")

    @functools.partial(
        pl.kernel,
        mesh=mesh,
        out_type=jax.ShapeDtypeStruct((total, DIM), jnp.float32),
        scratch_types=(
            [pltpu.VMEM((3, QBLK, CHUNK), jnp.int32)] * 2
            + [pltpu.VMEM((CHUNK, DIM), jnp.float32)] * (2 * RING)
            + [pltpu.SemaphoreType.DMA] * (3 * RING + 2)
        ),
    )
    def emb_kernel(w_ids, e_ids, w_tab, e_tab, out, *scratch):
        idxw, idxe = scratch[0], scratch[1]
        g1s = scratch[2:2 + RING]
        g2s = scratch[2 + RING:2 + 2 * RING]
        sems = scratch[2 + 2 * RING:]
        gwss = sems[0:RING]
        gess = sems[RING:2 * RING]
        wss = sems[2 * RING:3 * RING]
        ixw_sem, ixe_sem = sems[3 * RING], sems[3 * RING + 1]

        wid = lax.axis_index("s") * nc + lax.axis_index("c")
        cbase = wid * n_chunks
        n_blocks = n_chunks // QBLK

        def drain(sem, buf):
            pltpu.make_async_copy(w_tab.at[pl.ds(0, CHUNK)], buf, sem).wait()

        def start_load_idx(q):
            slot = lax.rem(q, 3)
            src = pl.ds(cbase + q * QBLK, QBLK)
            pltpu.async_copy(w_ids.at[src], idxw.at[slot], ixw_sem)
            pltpu.async_copy(e_ids.at[src], idxe.at[slot], ixe_sem)

        def wait_load_idx():
            pltpu.make_async_copy(w_ids.at[pl.ds(0, QBLK)], idxw.at[0],
                                  ixw_sem).wait()
            pltpu.make_async_copy(e_ids.at[pl.ds(0, QBLK)], idxe.at[0],
                                  ixe_sem).wait()

        def issue_gather(i, a):
            q = lax.div(i, QBLK)
            slot = lax.rem(q, 3)
            row = lax.rem(i, QBLK)
            pltpu.async_copy(w_tab.at[idxw.at[slot, row]], g1s[a], gwss[a])
            pltpu.async_copy(e_tab.at[idxe.at[slot, row]], g2s[a], gess[a])

        start_load_idx(0)
        wait_load_idx()
        start_load_idx(1)
        for p in range(DEPTH):
            issue_gather(p, p)

        def outer(i2, carry):
            for br in range(RING):
                i = RING * i2 + br
                g1, g2 = g1s[br], g2s[br]

                drain(gwss[br], g1)
                drain(gess[br], g2)

                @plsc.parallel_loop(0, CHUNK, unroll=2)
                def _(r):
                    for g in range(DIM // LANES):
                        sl = pl.ds(g * LANES, LANES)
                        plsc.addupdate(g1.at[r, sl], g2[r, sl])

                pltpu.async_copy(g1, out.at[pl.ds((cbase + i) * CHUNK, CHUNK)],
                                 wss[br])

                nxt = i + DEPTH
                na = (br + DEPTH) % RING

                def prefetch():
                    @pl.when(lax.rem(nxt, QBLK) == 0)
                    def _():
                        wait_load_idx()

                        @pl.when(lax.div(nxt, QBLK) + 1 < n_blocks)
                        def _():
                            start_load_idx(lax.div(nxt, QBLK) + 1)

                    issue_gather(nxt, na)

                if br >= RING - DEPTH:
                    @pl.when(nxt < n_chunks)
                    def _():
                        drain(wss[na], g1s[na])
                        prefetch()
                else:
                    @pl.when(nxt < n_chunks)
                    def _():
                        @pl.when(i2 >= 1)
                        def _():
                            drain(wss[na], g1s[na])

                        prefetch()
            return carry

        lax.fori_loop(0, n_chunks // RING, outer, 0)
        for a in range(RING):
            drain(wss[a], g1s[a])

    return emb_kernel


def kernel(word_ids, extword_ids, word_table, ext_table):
    b, l = word_ids.shape
    total = b * l
    w_2d = word_ids.reshape(total // CHUNK, CHUNK).astype(jnp.int32)
    e_2d = extword_ids.reshape(total // CHUNK, CHUNK).astype(jnp.int32)
    out = _build(total)(w_2d, e_2d, word_table, ext_table)
    return out.reshape(b, l, DIM)


# speedup vs baseline: 1.0048x; 1.0048x over previous
import functools

import jax
import jax.numpy as jnp
from jax import lax
from jax.experimental import pallas as pl
from jax.experimental.pallas import tpu as pltpu
from jax.experimental.pallas import tpu_sc as plsc

DIM = 128
CHUNK = 80
LANES = 16
QBLK = 32
RING = 5
DEPTH = 3


@functools.lru_cache(maxsize=None)
def _build(total):
    info = plsc.get_sparse_core_info()
    nc, ns = info.num_cores, info.num_subcores
    nw = nc * ns
    b_per_w = total // nw
    n_chunks = b_per_w // CHUNK
    assert total % (nw * CHUNK) == 0 and n_chunks % QBLK == 0
    assert n_chunks % RING == 0 and RING >= DEPTH + 2

    mesh = plsc.VectorSubcoreMesh(core_axis_name="c", subcore_axis_name="s")

    @functools.partial(
        pl.kernel,
        mesh=mesh,
        out_type=jax.ShapeDtypeStruct((total, DIM), jnp.float32),
        scratch_types=(
            [pltpu.VMEM((3, QBLK, CHUNK), jnp.int32)] * 2
            + [pltpu.VMEM((CHUNK, DIM), jnp.float32)] * (2 * RING)
            + [pltpu.SemaphoreType.DMA] * (3 * RING + 2)
        ),
    )
    def emb_kernel(w_ids, e_ids, w_tab, e_tab, out, *scratch):
        idxw, idxe = scratch[0], scratch[1]
        g1s = scratch[2:2 + RING]
        g2s = scratch[2 + RING:2 + 2 * RING]
        sems = scratch[2 + 2 * RING:]
        gwss = sems[0:RING]
        gess = sems[RING:2 * RING]
        wss = sems[2 * RING:3 * RING]
        ixw_sem, ixe_sem = sems[3 * RING], sems[3 * RING + 1]

        wid = lax.axis_index("s") * nc + lax.axis_index("c")
        cbase = wid * n_chunks
        n_blocks = n_chunks // QBLK

        def drain(sem, buf):
            pltpu.make_async_copy(w_tab.at[pl.ds(0, CHUNK)], buf, sem).wait()

        def start_load_idx(q):
            slot = lax.rem(q, 3)
            src = pl.ds(cbase + q * QBLK, QBLK)
            pltpu.async_copy(w_ids.at[src], idxw.at[slot], ixw_sem)
            pltpu.async_copy(e_ids.at[src], idxe.at[slot], ixe_sem)

        def wait_load_idx():
            pltpu.make_async_copy(w_ids.at[pl.ds(0, QBLK)], idxw.at[0],
                                  ixw_sem).wait()
            pltpu.make_async_copy(e_ids.at[pl.ds(0, QBLK)], idxe.at[0],
                                  ixe_sem).wait()

        def issue_gather(i, a):
            q = lax.div(i, QBLK)
            slot = lax.rem(q, 3)
            row = lax.rem(i, QBLK)
            pltpu.async_copy(w_tab.at[idxw.at[slot, row]], g1s[a], gwss[a])
            pltpu.async_copy(e_tab.at[idxe.at[slot, row]], g2s[a], gess[a])

        start_load_idx(0)
        wait_load_idx()
        start_load_idx(1)
        for p in range(DEPTH):
            issue_gather(p, p)

        def outer(i2, carry):
            for br in range(RING):
                i = RING * i2 + br
                g1, g2 = g1s[br], g2s[br]

                drain(gwss[br], g1)
                drain(gess[br], g2)

                def row_body(r, c):
                    for g in range(DIM // LANES):
                        sl = pl.ds(g * LANES, LANES)
                        plsc.addupdate(g1.at[r, sl], g2[r, sl])
                    return c

                lax.fori_loop(0, CHUNK, row_body, 0)

                pltpu.async_copy(g1, out.at[pl.ds((cbase + i) * CHUNK, CHUNK)],
                                 wss[br])

                nxt = i + DEPTH
                na = (br + DEPTH) % RING

                def prefetch():
                    @pl.when(lax.rem(nxt, QBLK) == 0)
                    def _():
                        wait_load_idx()

                        @pl.when(lax.div(nxt, QBLK) + 1 < n_blocks)
                        def _():
                            start_load_idx(lax.div(nxt, QBLK) + 1)

                    issue_gather(nxt, na)

                if br >= RING - DEPTH:
                    @pl.when(nxt < n_chunks)
                    def _():
                        drain(wss[na], g1s[na])
                        prefetch()
                else:
                    @pl.when(nxt < n_chunks)
                    def _():
                        @pl.when(i2 >= 1)
                        def _():
                            drain(wss[na], g1s[na])

                        prefetch()
            return carry

        lax.fori_loop(0, n_chunks // RING, outer, 0)
        for a in range(RING):
            drain(wss[a], g1s[a])

    return emb_kernel


def kernel(word_ids, extword_ids, word_table, ext_table):
    b, l = word_ids.shape
    total = b * l
    w_2d = word_ids.reshape(total // CHUNK, CHUNK).astype(jnp.int32)
    e_2d = extword_ids.reshape(total // CHUNK, CHUNK).astype(jnp.int32)
    out = _build(total)(w_2d, e_2d, word_table, ext_table)
    return out.reshape(b, l, DIM)
